# baseline (speedup 1.0000x reference)
import functools

import jax
import jax.numpy as jnp
from jax import lax
from jax.experimental import pallas as pl
from jax.experimental.pallas import tpu as pltpu
from jax.experimental.pallas import tpu_sc as plsc

_N = 10000
_NS = 16
_C = 128
_E = _N * _NS
_EPS = 1e-5

_NPAD = 10240
_CB = 1024
_NCHUNK = _NPAD // _CB
_QB = 128


def _full_spec(shape):
    return pl.BlockSpec(shape, lambda i, _s=shape: (0,) * len(_s))


def _qkv_body(x_ref, w_ref, b_ref, q_ref, k_ref, v_ref):
    out = jnp.dot(x_ref[...], w_ref[...],
                  preferred_element_type=jnp.float32) + b_ref[...]
    q_ref[...] = out[:, 0:_C]
    k_ref[...] = out[:, _C:2 * _C]
    v_ref[...] = out[:, 2 * _C:3 * _C]


def _qkv(x, W, b):
    RB = 400
    return pl.pallas_call(
        _qkv_body,
        grid=(_N // RB,),
        in_specs=[
            pl.BlockSpec((RB, _C), lambda i: (i, 0)),
            _full_spec((_C, 3 * _C)),
            _full_spec((1, 3 * _C)),
        ],
        out_specs=[pl.BlockSpec((RB, _C), lambda i: (i, 0))] * 3,
        out_shape=[jax.ShapeDtypeStruct((_N, _C), jnp.float32)] * 3,
    )(x, W, b)


_BIGF = 1e30


def _knnbuild_body(q_ref, pt_ref, s_ref, dm_ref, cnt_ref, d_ref):
    INF = jnp.float32(jnp.inf)

    def build(jb, M):
        pxyz = pt_ref[jb]
        dx = q_ref[:, 0:1] - pxyz[0:1, :]
        dy = q_ref[:, 1:2] - pxyz[1:2, :]
        dz = q_ref[:, 2:3] - pxyz[2:3, :]
        d = dx * dx + dy * dy + dz * dz
        d_ref[jb] = d
        dmin = jnp.min(d.reshape(_QB, _CB // 128, 128), axis=1)
        return jnp.minimum(M, dmin)

    M = lax.fori_loop(0, _NCHUNK, build,
                      jnp.full((_QB, 128), INF, jnp.float32))
    tau = None
    for _ in range(_NS):
        tau = jnp.min(M, axis=1, keepdims=True)
        M = jnp.where(M == tau, INF, M)

    i = pl.program_id(0)
    rowid = lax.broadcasted_iota(jnp.int32, (_QB, 1), 0) + i * _QB
    rowok = rowid < _N
    for jb in range(_NCHUNK):
        dblk = d_ref[jb]
        keep = (dblk <= tau) & rowok
        dm_ref[:, jb, :] = jnp.where(keep, dblk, _BIGF)
        kcnt = jnp.dot(keep.astype(jnp.float32), s_ref[...],
                       preferred_element_type=jnp.float32)
        cnt_ref[:, jb * (_CB // 16):(jb + 1) * (_CB // 16)] = (
            kcnt.astype(jnp.int32))


def _knnbuild(p_pad, pt3, smat):
    return pl.pallas_call(
        _knnbuild_body,
        grid=(_NPAD // _QB,),
        in_specs=[
            pl.BlockSpec((_QB, 8), lambda i: (i, 0)),
            _full_spec((_NCHUNK, 8, _CB)),
            _full_spec((_CB, _CB // 16)),
        ],
        out_specs=[pl.BlockSpec((_QB, _NCHUNK, _CB), lambda i: (i, 0, 0)),
                   pl.BlockSpec((_QB, _NPAD // 16), lambda i: (i, 0))],
        out_shape=[jax.ShapeDtypeStruct((_NPAD, _NCHUNK, _CB), jnp.float32),
                   jax.ShapeDtypeStruct((_NPAD, _NPAD // 16), jnp.int32)],
        scratch_shapes=[pltpu.VMEM((_NCHUNK, _QB, _CB), jnp.float32)],
    )(p_pad, pt3, smat)


def _scselect(dm2, cnts):
    info = plsc.get_sparse_core_info()
    nw = info.num_cores * info.num_subcores
    rows_pw = _NPAD // nw
    NGC = _NPAD // 16

    _gdn = lax.GatherDimensionNumbers(offset_dims=(),
                                      collapsed_slice_dims=(0,),
                                      start_index_map=(0,))

    def _gat(a, iv):
        return lax.gather(a, iv[:, None], _gdn, slice_sizes=(1,),
                          mode=lax.GatherScatterMode.PROMISE_IN_BOUNDS)
    NG4 = NGC // 4
    mesh = plsc.VectorSubcoreMesh(core_axis_name="c", subcore_axis_name="s")
    INF = jnp.float32(jnp.inf)
    ISENT = jnp.int32(2 ** 30)

    @functools.partial(
        pl.kernel,
        out_type=jax.ShapeDtypeStruct((_NPAD * _NS,), jnp.int32),
        mesh=mesh,
        scratch_types=[pltpu.VMEM((_NPAD,), jnp.float32),
                       pltpu.VMEM((NGC,), jnp.int32),
                       pltpu.VMEM((16,), jnp.float32),
                       pltpu.VMEM((16,), jnp.int32),
                       pltpu.VMEM((rows_pw * _NS,), jnp.int32),
                       pltpu.SemaphoreType.DMA],
    )
    def sel(dm_hbm, cnt_hbm, out_hbm, rbuf, cbuf, accd_ref, acci_ref, ob,
            s0):
        wid = lax.axis_index("s") * info.num_cores + lax.axis_index("c")
        base = wid * rows_pw

        def row_fn(r, _):
            row = base + r
            pltpu.sync_copy(dm_hbm.at[row], rbuf)
            pltpu.sync_copy(cnt_hbm.at[row], cbuf)
            accd_ref[...] = jnp.full((16,), INF, jnp.float32)
            acci_ref[...] = jnp.full((16,), ISENT, jnp.int32)

            def scan_group(g16, _ig):
                cv = cbuf[pl.ds(g16 * 16, 16)]
                tot = cv[0]
                for k in range(1, 16):
                    tot = tot + cv[k]

                shift_iv = jnp.maximum(
                    jnp.arange(16, dtype=jnp.int32) - 1, 0)
                lane0 = jnp.arange(16, dtype=jnp.int32) == 0

                @pl.when(tot > 0)
                def _slow():
                    for k in range(16):
                        @pl.when(cv[k] > 0)
                        def _hit(_k=k):
                            g = g16 * 16 + _k
                            v = rbuf[pl.ds(g * 16, 16)]
                            for l in range(16):
                                @pl.when(v[l] < _BIGF)
                                def _ins(_l=l):
                                    c = v[_l]
                                    cpos = g * 16 + _l
                                    ad = accd_ref[...]
                                    ai = acci_ref[...]
                                    shd = _gat(ad, shift_iv)
                                    shi = _gat(ai, shift_iv)
                                    shd = jnp.where(lane0,
                                                    jnp.float32(-1.0), shd)
                                    le = ad <= c
                                    shle = shd <= c
                                    nd = jnp.where(
                                        le, ad, jnp.where(shle, c, shd))
                                    ni = jnp.where(
                                        le, ai, jnp.where(shle, cpos, shi))
                                    accd_ref[...] = nd
                                    acci_ref[...] = ni

                return 0

            lax.fori_loop(0, NGC // 16, scan_group, 0)
            ob[pl.ds(r * _NS, 16)] = acci_ref[...]
            return 0

        lax.fori_loop(0, rows_pw, row_fn, 0)
        pltpu.sync_copy(ob, out_hbm.at[pl.ds(base * _NS, rows_pw * _NS)])

    return sel(dm2, cnts)


def _gather3(idx_flat, tk, tv, tp):
    info = plsc.get_sparse_core_info()
    nw = info.num_cores * info.num_subcores
    per_w = _E // nw
    CH = 40
    nchunk = per_w // CH
    mesh = plsc.VectorSubcoreMesh(core_axis_name="c", subcore_axis_name="s")

    @functools.partial(
        pl.kernel,
        out_type=[jax.ShapeDtypeStruct((_E, _C), jnp.float32),
                  jax.ShapeDtypeStruct((_E, _C), jnp.float32),
                  jax.ShapeDtypeStruct((_E, _C), jnp.float32)],
        mesh=mesh,
        scratch_types=[pltpu.VMEM((CH,), jnp.int32),
                       pltpu.VMEM((CH, _C), jnp.float32),
                       pltpu.VMEM((CH, _C), jnp.float32),
                       pltpu.VMEM((CH, _C), jnp.float32),
                       pltpu.SemaphoreType.DMA,
                       pltpu.SemaphoreType.DMA,
                       pltpu.SemaphoreType.DMA],
    )
    def gath(idx_hbm, tk_hbm, tv_hbm, tp_hbm, ok_hbm, ov_hbm, op_hbm,
             idx_v, bk, bv, bp, s1, s2, s3):
        wid = lax.axis_index("s") * info.num_cores + lax.axis_index("c")
        base = wid * per_w

        def chunk(cc, _):
            start = base + cc * CH
            pltpu.sync_copy(idx_hbm.at[pl.ds(start, CH)], idx_v)
            c1 = pltpu.async_copy(tk_hbm.at[idx_v], bk, s1)
            c2 = pltpu.async_copy(tv_hbm.at[idx_v], bv, s2)
            c3 = pltpu.async_copy(tp_hbm.at[idx_v], bp, s3)
            c1.wait()
            c2.wait()
            c3.wait()
            pltpu.sync_copy(bk, ok_hbm.at[pl.ds(start, CH)])
            pltpu.sync_copy(bv, ov_hbm.at[pl.ds(start, CH)])
            pltpu.sync_copy(bp, op_hbm.at[pl.ds(start, CH)])
            return 0

        lax.fori_loop(0, nchunk, chunk, 0)

    return gath(idx_flat, tk, tv, tp)


def _tstats_body(pp_ref, pc_ref, w_ref, b_ref, o_ref):
    @pl.when(pl.program_id(0) == 0)
    def _():
        o_ref[...] = jnp.zeros_like(o_ref)

    bn = pp_ref.shape[0]
    pr = (pp_ref[...] - jnp.broadcast_to(pc_ref[...], (bn, _NS, _C))
          ).reshape(bn * _NS, _C)
    t = jnp.dot(pr, w_ref[...],
                preferred_element_type=jnp.float32) + b_ref[...]
    o_ref[0:1, :] += jnp.sum(t, axis=0, keepdims=True)
    o_ref[1:2, :] += jnp.sum(t * t, axis=0, keepdims=True)


def _tstats(pp_e3, pc3, Wp1p, bp1p):
    BT = 125
    return pl.pallas_call(
        _tstats_body,
        grid=(_N // BT,),
        in_specs=[
            pl.BlockSpec((BT, _NS, _C), lambda i: (i, 0, 0)),
            pl.BlockSpec((BT, 1, _C), lambda i: (i, 0, 0)),
            _full_spec((_C, 16)),
            _full_spec((1, 16)),
        ],
        out_specs=_full_spec((8, 16)),
        out_shape=jax.ShapeDtypeStruct((8, 16), jnp.float32),
    )(pp_e3, pc3, Wp1p, bp1p)


_BN = 80
_GRID_E = _N // _BN


def _front(pp_ref, pc_ref, xk_ref, q_ref, wp1_ref, bp1_ref, at_ref, bt_ref,
           wp2_ref, bp2_ref):
    pr = (pp_ref[...] - jnp.broadcast_to(pc_ref[...], (_BN, _NS, _C))
          ).reshape(_BN * _NS, _C)
    t = jnp.dot(pr, wp1_ref[...],
                preferred_element_type=jnp.float32) + bp1_ref[...]
    h = jnp.maximum(t * at_ref[...] + bt_ref[...], 0.0)
    pe = jnp.dot(h, wp2_ref[...],
                 preferred_element_type=jnp.float32) + bp2_ref[...]
    xk = xk_ref[...].reshape(_BN * _NS, _C)
    q = jnp.broadcast_to(q_ref[...], (_BN, _NS, _C)).reshape(_BN * _NS, _C)
    return pe, xk - q + pe


def _w0stats_body(pp_ref, pc_ref, xk_ref, q_ref, wp1_ref, bp1_ref, at_ref,
                  bt_ref, wp2_ref, bp2_ref, o_ref):
    @pl.when(pl.program_id(0) == 0)
    def _():
        o_ref[...] = jnp.zeros_like(o_ref)

    _, w0 = _front(pp_ref, pc_ref, xk_ref, q_ref, wp1_ref, bp1_ref, at_ref,
                   bt_ref, wp2_ref, bp2_ref)
    o_ref[0:1, :] += jnp.sum(w0, axis=0, keepdims=True)
    o_ref[1:2, :] += jnp.sum(w0 * w0, axis=0, keepdims=True)


def _w0stats(pp_e3, pc3, xk_e3, q3, *params):
    return pl.pallas_call(
        _w0stats_body,
        grid=(_GRID_E,),
        in_specs=[
            pl.BlockSpec((_BN, _NS, _C), lambda i: (i, 0, 0)),
            pl.BlockSpec((_BN, 1, _C), lambda i: (i, 0, 0)),
            pl.BlockSpec((_BN, _NS, _C), lambda i: (i, 0, 0)),
            pl.BlockSpec((_BN, 1, _C), lambda i: (i, 0, 0)),
            _full_spec((_C, 16)),
            _full_spec((1, 16)),
            _full_spec((1, 16)),
            _full_spec((1, 16)),
            _full_spec((16, _C)),
            _full_spec((1, _C)),
        ],
        out_specs=_full_spec((8, _C)),
        out_shape=jax.ShapeDtypeStruct((8, _C), jnp.float32),
    )(pp_e3, pc3, xk_e3, q3, *params)


def _w1_body(pp_ref, pc_ref, xk_ref, q_ref, wp1_ref, bp1_ref, at_ref, bt_ref,
             wp2_ref, bp2_ref, aw_ref, bw_ref, ww1_ref, bw1_ref,
             w1_ref, o_ref):
    @pl.when(pl.program_id(0) == 0)
    def _():
        o_ref[...] = jnp.zeros_like(o_ref)

    _, w0 = _front(pp_ref, pc_ref, xk_ref, q_ref, wp1_ref, bp1_ref, at_ref,
                   bt_ref, wp2_ref, bp2_ref)
    h1 = jnp.maximum(w0 * aw_ref[...] + bw_ref[...], 0.0)
    w1 = jnp.dot(h1, ww1_ref[...],
                 preferred_element_type=jnp.float32) + bw1_ref[...]
    w1_ref[...] = w1.reshape(_BN, _NS, 16)
    o_ref[0:1, :] += jnp.sum(w1, axis=0, keepdims=True)
    o_ref[1:2, :] += jnp.sum(w1 * w1, axis=0, keepdims=True)


def _w1pass(pp_e3, pc3, xk_e3, q3, *params):
    return pl.pallas_call(
        _w1_body,
        grid=(_GRID_E,),
        in_specs=[
            pl.BlockSpec((_BN, _NS, _C), lambda i: (i, 0, 0)),
            pl.BlockSpec((_BN, 1, _C), lambda i: (i, 0, 0)),
            pl.BlockSpec((_BN, _NS, _C), lambda i: (i, 0, 0)),
            pl.BlockSpec((_BN, 1, _C), lambda i: (i, 0, 0)),
            _full_spec((_C, 16)),
            _full_spec((1, 16)),
            _full_spec((1, 16)),
            _full_spec((1, 16)),
            _full_spec((16, _C)),
            _full_spec((1, _C)),
            _full_spec((1, _C)),
            _full_spec((1, _C)),
            _full_spec((_C, 16)),
            _full_spec((1, 16)),
        ],
        out_specs=[pl.BlockSpec((_BN, _NS, 16), lambda i: (i, 0, 0)),
                   _full_spec((8, 16))],
        out_shape=[jax.ShapeDtypeStruct((_N, _NS, 16), jnp.float32),
                   jax.ShapeDtypeStruct((8, 16), jnp.float32)],
    )(pp_e3, pc3, xk_e3, q3, *params)


def _final_body(w1_ref, xv_ref, pp_ref, pc_ref, wp1_ref, bp1_ref, at_ref,
                bt_ref, wp2_ref, bp2_ref, a2_ref, b2_ref, ww2_ref, bw2_ref,
                tmat_ref, o_ref):
    w1 = w1_ref[...].reshape(_BN * _NS, 16)
    h2 = jnp.maximum(w1 * a2_ref[...] + b2_ref[...], 0.0)
    w2 = (jnp.dot(h2, ww2_ref[...], preferred_element_type=jnp.float32)
          + bw2_ref[...]).reshape(_BN, _NS, 16)
    mx = jnp.max(w2, axis=1, keepdims=True)
    e = jnp.exp(w2 - mx)
    sm = e / jnp.sum(e, axis=1, keepdims=True)
    wfull = jnp.dot(sm.reshape(_BN * _NS, 16), tmat_ref[...],
                    preferred_element_type=jnp.float32
                    ).reshape(_BN, _NS, _C)
    pr = (pp_ref[...] - jnp.broadcast_to(pc_ref[...], (_BN, _NS, _C))
          ).reshape(_BN * _NS, _C)
    t = jnp.dot(pr, wp1_ref[...],
                preferred_element_type=jnp.float32) + bp1_ref[...]
    h = jnp.maximum(t * at_ref[...] + bt_ref[...], 0.0)
    pe = (jnp.dot(h, wp2_ref[...], preferred_element_type=jnp.float32)
          + bp2_ref[...]).reshape(_BN, _NS, _C)
    o_ref[...] = jnp.sum((xv_ref[...] + pe) * wfull, axis=1)


def _final(w1_e3, xv_e3, pp_e3, pc3, *params):
    return pl.pallas_call(
        _final_body,
        grid=(_GRID_E,),
        in_specs=[
            pl.BlockSpec((_BN, _NS, 16), lambda i: (i, 0, 0)),
            pl.BlockSpec((_BN, _NS, _C), lambda i: (i, 0, 0)),
            pl.BlockSpec((_BN, _NS, _C), lambda i: (i, 0, 0)),
            pl.BlockSpec((_BN, 1, _C), lambda i: (i, 0, 0)),
            _full_spec((_C, 16)),
            _full_spec((1, 16)),
            _full_spec((1, 16)),
            _full_spec((1, 16)),
            _full_spec((16, _C)),
            _full_spec((1, _C)),
            _full_spec((1, 16)),
            _full_spec((1, 16)),
            _full_spec((16, 16)),
            _full_spec((1, 16)),
            _full_spec((16, _C)),
        ],
        out_specs=pl.BlockSpec((_BN, _C), lambda i: (i, 0)),
        out_shape=jax.ShapeDtypeStruct((_N, _C), jnp.float32),
    )(w1_e3, xv_e3, pp_e3, pc3, *params)


def kernel(p, x, o, Wq, bq, Wk, bk, Wv, bv, Wp1, bp1, g_p, be_p, Wp2, bp2,
           g_w1, be_w1, Ww1, bw1, g_w2, be_w2, Ww2, bw2):
    f32 = jnp.float32
    W = jnp.concatenate([Wq, Wk, Wv], axis=1)
    b = jnp.concatenate([bq, bk, bv])[None, :]
    q, xk, xv = _qkv(x, W, b)

    p8 = jnp.pad(p, ((0, 0), (0, 5)))
    p_pad = jnp.pad(p8, ((0, _NPAD - _N), (0, 0)), constant_values=1e9)
    pt3 = p_pad.T.reshape(8, _NCHUNK, _CB).transpose(1, 0, 2)
    smat = (lax.broadcasted_iota(jnp.int32, (_CB, _CB // 16), 0) // 16 ==
            lax.broadcasted_iota(jnp.int32, (_CB, _CB // 16), 1)
            ).astype(f32)
    dm3, cnts = _knnbuild(p_pad, pt3, smat)
    idx_all = _scselect(dm3.reshape(_NPAD, _NPAD), cnts)
    idx_flat = idx_all.reshape(_NPAD, _NS)[:_N].reshape(-1)

    pp128 = jnp.pad(p, ((0, 0), (0, _C - 3)))
    xk_e, xv_e, pp_e = _gather3(idx_flat, xk, xv, pp128)

    Wp1p = jnp.zeros((_C, 16), f32).at[:3, :3].set(Wp1)
    bp1p = jnp.zeros((1, 16), f32).at[0, :3].set(bp1)
    M = float(_E)
    pp_e3 = pp_e.reshape(_N, _NS, _C)
    pc3 = pp128.reshape(_N, 1, _C)
    ts = _tstats(pp_e3, pc3, Wp1p, bp1p)
    mu_t = ts[0] / M
    var_t = ts[1] / M - mu_t * mu_t
    gp = jnp.zeros((16,), f32).at[:3].set(g_p)
    bep = jnp.zeros((16,), f32).at[:3].set(be_p)
    a_t = (gp * lax.rsqrt(var_t + _EPS))[None, :]
    b_t = bep[None, :] - mu_t[None, :] * a_t

    Wp2p = jnp.zeros((16, _C), f32).at[:3, :].set(Wp2)
    bp2r = bp2[None, :]
    xk_e3 = xk_e.reshape(_N, _NS, _C)
    xv_e3 = xv_e.reshape(_N, _NS, _C)
    q3 = q.reshape(_N, 1, _C)
    front_params = (Wp1p, bp1p, a_t, b_t, Wp2p, bp2r)

    ws = _w0stats(pp_e3, pc3, xk_e3, q3, *front_params)
    mu_w = ws[0] / M
    var_w = ws[1] / M - mu_w * mu_w
    a_w = (g_w1 * lax.rsqrt(var_w + _EPS))[None, :]
    b_w = be_w1[None, :] - mu_w[None, :] * a_w

    w1_e3, w1s = _w1pass(pp_e3, pc3, xk_e3, q3, *front_params,
                         a_w, b_w, Ww1, bw1[None, :])
    mu1 = w1s[0] / M
    var1 = w1s[1] / M - mu1 * mu1
    a_2 = (g_w2 * lax.rsqrt(var1 + _EPS))[None, :]
    b_2 = be_w2[None, :] - mu1[None, :] * a_2

    tmat = (lax.broadcasted_iota(jnp.int32, (16, _C), 1) % 16 ==
            lax.broadcasted_iota(jnp.int32, (16, _C), 0)).astype(f32)
    out = _final(w1_e3, xv_e3, pp_e3, pc3, *front_params,
                 a_2, b_2, Ww2, bw2[None, :], tmat)
    return out


# speedup vs baseline: 4.6194x; 4.6194x over previous
import functools

import jax
import jax.numpy as jnp
from jax import lax
from jax.experimental import pallas as pl
from jax.experimental.pallas import tpu as pltpu
from jax.experimental.pallas import tpu_sc as plsc

_N = 10000
_NS = 16
_C = 128
_E = _N * _NS
_EPS = 1e-5

_NPAD = 10240
_CB = 1024
_NCHUNK = _NPAD // _CB
_QB = 128


def _full_spec(shape):
    return pl.BlockSpec(shape, lambda i, _s=shape: (0,) * len(_s))


def _qkv_body(x_ref, w_ref, b_ref, q_ref, k_ref, v_ref):
    out = jnp.dot(x_ref[...], w_ref[...],
                  preferred_element_type=jnp.float32) + b_ref[...]
    q_ref[...] = out[:, 0:_C]
    k_ref[...] = out[:, _C:2 * _C]
    v_ref[...] = out[:, 2 * _C:3 * _C]


def _qkv(x, W, b):
    RB = 400
    return pl.pallas_call(
        _qkv_body,
        grid=(_N // RB,),
        in_specs=[
            pl.BlockSpec((RB, _C), lambda i: (i, 0)),
            _full_spec((_C, 3 * _C)),
            _full_spec((1, 3 * _C)),
        ],
        out_specs=[pl.BlockSpec((RB, _C), lambda i: (i, 0))] * 3,
        out_shape=[jax.ShapeDtypeStruct((_N, _C), jnp.float32)] * 3,
    )(x, W, b)


_L = 8


def _knn_body(q_ref, pt_ref, idx_ref, d_ref, cd_ref, cj_ref):
    BIG = jnp.int32(2 ** 30)
    INF = jnp.float32(jnp.inf)

    def build(jb, _):
        pxyz = pt_ref[jb]
        dx = q_ref[:, 0:1] - pxyz[0:1, :]
        dy = q_ref[:, 1:2] - pxyz[1:2, :]
        dz = q_ref[:, 2:3] - pxyz[2:3, :]
        d_ref[jb] = dx * dx + dy * dy + dz * dz
        return 0

    lax.fori_loop(0, _NCHUNK, build, 0)

    def p1(jb, _):
        iota = lax.broadcasted_iota(jnp.int32, (_QB, _CB), 1) + jb * _CB
        jp = jnp.full((_QB, 1), -1, jnp.int32)
        bms, bjs = [], []
        for t in range(_L):
            blk = jnp.where(iota == jp, INF, d_ref[jb])
            d_ref[jb] = blk
            bm = jnp.min(blk, axis=1, keepdims=True)
            bj = jnp.min(jnp.where(blk == bm, iota, BIG),
                         axis=1, keepdims=True)
            bms.append(bm)
            bjs.append(bj)
            jp = bj
        d_ref[jb] = jnp.where(iota == jp, INF, d_ref[jb])
        cd_ref[jb] = jnp.concatenate(bms, axis=1)
        cj_ref[jb] = jnp.concatenate(bjs, axis=1)
        return 0

    lax.fori_loop(0, _NCHUNK, p1, 0)

    cdv = jnp.concatenate([cd_ref[jb] for jb in range(_NCHUNK)], axis=1)
    cjv = jnp.concatenate([cj_ref[jb] for jb in range(_NCHUNK)], axis=1)
    lane = lax.broadcasted_iota(jnp.int32, (_QB, _NCHUNK * _L), 1)
    tau = None
    for t in range(_NS):
        m = jnp.min(cdv, axis=1, keepdims=True)
        j = jnp.min(jnp.where(cdv == m, cjv, BIG), axis=1, keepdims=True)
        lsel = jnp.min(jnp.where((cdv == m) & (cjv == j), lane, BIG),
                       axis=1, keepdims=True)
        cdv = jnp.where(lane == lsel, INF, cdv)
        idx_ref[:, t:t + 1] = j
        tau = m

    def verify(jb, acc):
        bm = jnp.min(d_ref[jb], axis=1, keepdims=True)
        return acc + (bm <= tau).astype(jnp.int32)

    bad = lax.fori_loop(0, _NCHUNK, verify,
                        jnp.zeros((_QB, 1), jnp.int32))
    nbad = jnp.sum(bad)

    @pl.when(nbad > 0)
    def _fallback():
        lax.fori_loop(0, _NCHUNK, build, 0)
        jprev = jnp.full((_QB, 1), -1, jnp.int32)
        for t in range(_NS):
            def fmin(jb, carry):
                am, aj, jp = carry
                iota = (lax.broadcasted_iota(jnp.int32, (_QB, _CB), 1)
                        + jb * _CB)
                blk = jnp.where(iota == jp, INF, d_ref[jb])
                d_ref[jb] = blk
                bm = jnp.min(blk, axis=1, keepdims=True)
                bj = jnp.min(jnp.where(blk == bm, iota, BIG),
                             axis=1, keepdims=True)
                take = bm < am
                return (jnp.where(take, bm, am), jnp.where(take, bj, aj),
                        jp)

            m, j, _ = lax.fori_loop(
                0, _NCHUNK, fmin,
                (jnp.full((_QB, 1), INF, jnp.float32),
                 jnp.full((_QB, 1), BIG, jnp.int32),
                 jprev))
            idx_ref[:, t:t + 1] = j
            jprev = j


def _knn(p_pad, pt3):
    return pl.pallas_call(
        _knn_body,
        grid=(_NPAD // _QB,),
        in_specs=[
            pl.BlockSpec((_QB, 8), lambda i: (i, 0)),
            _full_spec((_NCHUNK, 8, _CB)),
        ],
        out_specs=pl.BlockSpec((_QB, _NS), lambda i: (i, 0)),
        out_shape=jax.ShapeDtypeStruct((_NPAD, _NS), jnp.int32),
        scratch_shapes=[pltpu.VMEM((_NCHUNK, _QB, _CB), jnp.float32),
                        pltpu.VMEM((_NCHUNK, _QB, _L), jnp.float32),
                        pltpu.VMEM((_NCHUNK, _QB, _L), jnp.int32)],
    )(p_pad, pt3)


def _gather3(idx_flat, tk, tv, tp):
    info = plsc.get_sparse_core_info()
    nw = info.num_cores * info.num_subcores
    per_w = _E // nw
    CH = 40
    nchunk = per_w // CH
    mesh = plsc.VectorSubcoreMesh(core_axis_name="c", subcore_axis_name="s")

    @functools.partial(
        pl.kernel,
        out_type=[jax.ShapeDtypeStruct((_E, _C), jnp.float32),
                  jax.ShapeDtypeStruct((_E, _C), jnp.float32),
                  jax.ShapeDtypeStruct((_E, _C), jnp.float32)],
        mesh=mesh,
        scratch_types=[pltpu.VMEM((CH,), jnp.int32),
                       pltpu.VMEM((CH, _C), jnp.float32),
                       pltpu.VMEM((CH, _C), jnp.float32),
                       pltpu.VMEM((CH, _C), jnp.float32),
                       pltpu.SemaphoreType.DMA,
                       pltpu.SemaphoreType.DMA,
                       pltpu.SemaphoreType.DMA],
    )
    def gath(idx_hbm, tk_hbm, tv_hbm, tp_hbm, ok_hbm, ov_hbm, op_hbm,
             idx_v, bk, bv, bp, s1, s2, s3):
        wid = lax.axis_index("s") * info.num_cores + lax.axis_index("c")
        base = wid * per_w

        def chunk(cc, _):
            start = base + cc * CH
            pltpu.sync_copy(idx_hbm.at[pl.ds(start, CH)], idx_v)
            c1 = pltpu.async_copy(tk_hbm.at[idx_v], bk, s1)
            c2 = pltpu.async_copy(tv_hbm.at[idx_v], bv, s2)
            c3 = pltpu.async_copy(tp_hbm.at[idx_v], bp, s3)
            c1.wait()
            c2.wait()
            c3.wait()
            pltpu.sync_copy(bk, ok_hbm.at[pl.ds(start, CH)])
            pltpu.sync_copy(bv, ov_hbm.at[pl.ds(start, CH)])
            pltpu.sync_copy(bp, op_hbm.at[pl.ds(start, CH)])
            return 0

        lax.fori_loop(0, nchunk, chunk, 0)

    return gath(idx_flat, tk, tv, tp)


def _tstats_body(pp_ref, pc_ref, w_ref, b_ref, o_ref):
    @pl.when(pl.program_id(0) == 0)
    def _():
        o_ref[...] = jnp.zeros_like(o_ref)

    bn = pp_ref.shape[0]
    pr = (pp_ref[...] - jnp.broadcast_to(pc_ref[...], (bn, _NS, _C))
          ).reshape(bn * _NS, _C)
    t = jnp.dot(pr, w_ref[...],
                preferred_element_type=jnp.float32) + b_ref[...]
    o_ref[0:1, :] += jnp.sum(t, axis=0, keepdims=True)
    o_ref[1:2, :] += jnp.sum(t * t, axis=0, keepdims=True)


def _tstats(pp_e3, pc3, Wp1p, bp1p):
    BT = 125
    return pl.pallas_call(
        _tstats_body,
        grid=(_N // BT,),
        in_specs=[
            pl.BlockSpec((BT, _NS, _C), lambda i: (i, 0, 0)),
            pl.BlockSpec((BT, 1, _C), lambda i: (i, 0, 0)),
            _full_spec((_C, 16)),
            _full_spec((1, 16)),
        ],
        out_specs=_full_spec((8, 16)),
        out_shape=jax.ShapeDtypeStruct((8, 16), jnp.float32),
    )(pp_e3, pc3, Wp1p, bp1p)


_BN = 80
_GRID_E = _N // _BN


def _front(pp_ref, pc_ref, xk_ref, q_ref, wp1_ref, bp1_ref, at_ref, bt_ref,
           wp2_ref, bp2_ref):
    pr = (pp_ref[...] - jnp.broadcast_to(pc_ref[...], (_BN, _NS, _C))
          ).reshape(_BN * _NS, _C)
    t = jnp.dot(pr, wp1_ref[...],
                preferred_element_type=jnp.float32) + bp1_ref[...]
    h = jnp.maximum(t * at_ref[...] + bt_ref[...], 0.0)
    pe = jnp.dot(h, wp2_ref[...],
                 preferred_element_type=jnp.float32) + bp2_ref[...]
    xk = xk_ref[...].reshape(_BN * _NS, _C)
    q = jnp.broadcast_to(q_ref[...], (_BN, _NS, _C)).reshape(_BN * _NS, _C)
    return pe, xk - q + pe


def _w0stats_body(pp_ref, pc_ref, xk_ref, q_ref, wp1_ref, bp1_ref, at_ref,
                  bt_ref, wp2_ref, bp2_ref, o_ref):
    @pl.when(pl.program_id(0) == 0)
    def _():
        o_ref[...] = jnp.zeros_like(o_ref)

    _, w0 = _front(pp_ref, pc_ref, xk_ref, q_ref, wp1_ref, bp1_ref, at_ref,
                   bt_ref, wp2_ref, bp2_ref)
    o_ref[0:1, :] += jnp.sum(w0, axis=0, keepdims=True)
    o_ref[1:2, :] += jnp.sum(w0 * w0, axis=0, keepdims=True)


def _w0stats(pp_e3, pc3, xk_e3, q3, *params):
    return pl.pallas_call(
        _w0stats_body,
        grid=(_GRID_E,),
        in_specs=[
            pl.BlockSpec((_BN, _NS, _C), lambda i: (i, 0, 0)),
            pl.BlockSpec((_BN, 1, _C), lambda i: (i, 0, 0)),
            pl.BlockSpec((_BN, _NS, _C), lambda i: (i, 0, 0)),
            pl.BlockSpec((_BN, 1, _C), lambda i: (i, 0, 0)),
            _full_spec((_C, 16)),
            _full_spec((1, 16)),
            _full_spec((1, 16)),
            _full_spec((1, 16)),
            _full_spec((16, _C)),
            _full_spec((1, _C)),
        ],
        out_specs=_full_spec((8, _C)),
        out_shape=jax.ShapeDtypeStruct((8, _C), jnp.float32),
    )(pp_e3, pc3, xk_e3, q3, *params)


def _w1_body(pp_ref, pc_ref, xk_ref, q_ref, wp1_ref, bp1_ref, at_ref, bt_ref,
             wp2_ref, bp2_ref, aw_ref, bw_ref, ww1_ref, bw1_ref,
             w1_ref, o_ref):
    @pl.when(pl.program_id(0) == 0)
    def _():
        o_ref[...] = jnp.zeros_like(o_ref)

    _, w0 = _front(pp_ref, pc_ref, xk_ref, q_ref, wp1_ref, bp1_ref, at_ref,
                   bt_ref, wp2_ref, bp2_ref)
    h1 = jnp.maximum(w0 * aw_ref[...] + bw_ref[...], 0.0)
    w1 = jnp.dot(h1, ww1_ref[...],
                 preferred_element_type=jnp.float32) + bw1_ref[...]
    w1_ref[...] = w1.reshape(_BN, _NS, 16)
    o_ref[0:1, :] += jnp.sum(w1, axis=0, keepdims=True)
    o_ref[1:2, :] += jnp.sum(w1 * w1, axis=0, keepdims=True)


def _w1pass(pp_e3, pc3, xk_e3, q3, *params):
    return pl.pallas_call(
        _w1_body,
        grid=(_GRID_E,),
        in_specs=[
            pl.BlockSpec((_BN, _NS, _C), lambda i: (i, 0, 0)),
            pl.BlockSpec((_BN, 1, _C), lambda i: (i, 0, 0)),
            pl.BlockSpec((_BN, _NS, _C), lambda i: (i, 0, 0)),
            pl.BlockSpec((_BN, 1, _C), lambda i: (i, 0, 0)),
            _full_spec((_C, 16)),
            _full_spec((1, 16)),
            _full_spec((1, 16)),
            _full_spec((1, 16)),
            _full_spec((16, _C)),
            _full_spec((1, _C)),
            _full_spec((1, _C)),
            _full_spec((1, _C)),
            _full_spec((_C, 16)),
            _full_spec((1, 16)),
        ],
        out_specs=[pl.BlockSpec((_BN, _NS, 16), lambda i: (i, 0, 0)),
                   _full_spec((8, 16))],
        out_shape=[jax.ShapeDtypeStruct((_N, _NS, 16), jnp.float32),
                   jax.ShapeDtypeStruct((8, 16), jnp.float32)],
    )(pp_e3, pc3, xk_e3, q3, *params)


def _final_body(w1_ref, xv_ref, pp_ref, pc_ref, wp1_ref, bp1_ref, at_ref,
                bt_ref, wp2_ref, bp2_ref, a2_ref, b2_ref, ww2_ref, bw2_ref,
                tmat_ref, o_ref):
    w1 = w1_ref[...].reshape(_BN * _NS, 16)
    h2 = jnp.maximum(w1 * a2_ref[...] + b2_ref[...], 0.0)
    w2 = (jnp.dot(h2, ww2_ref[...], preferred_element_type=jnp.float32)
          + bw2_ref[...]).reshape(_BN, _NS, 16)
    mx = jnp.max(w2, axis=1, keepdims=True)
    e = jnp.exp(w2 - mx)
    sm = e / jnp.sum(e, axis=1, keepdims=True)
    wfull = jnp.dot(sm.reshape(_BN * _NS, 16), tmat_ref[...],
                    preferred_element_type=jnp.float32
                    ).reshape(_BN, _NS, _C)
    pr = (pp_ref[...] - jnp.broadcast_to(pc_ref[...], (_BN, _NS, _C))
          ).reshape(_BN * _NS, _C)
    t = jnp.dot(pr, wp1_ref[...],
                preferred_element_type=jnp.float32) + bp1_ref[...]
    h = jnp.maximum(t * at_ref[...] + bt_ref[...], 0.0)
    pe = (jnp.dot(h, wp2_ref[...], preferred_element_type=jnp.float32)
          + bp2_ref[...]).reshape(_BN, _NS, _C)
    o_ref[...] = jnp.sum((xv_ref[...] + pe) * wfull, axis=1)


def _final(w1_e3, xv_e3, pp_e3, pc3, *params):
    return pl.pallas_call(
        _final_body,
        grid=(_GRID_E,),
        in_specs=[
            pl.BlockSpec((_BN, _NS, 16), lambda i: (i, 0, 0)),
            pl.BlockSpec((_BN, _NS, _C), lambda i: (i, 0, 0)),
            pl.BlockSpec((_BN, _NS, _C), lambda i: (i, 0, 0)),
            pl.BlockSpec((_BN, 1, _C), lambda i: (i, 0, 0)),
            _full_spec((_C, 16)),
            _full_spec((1, 16)),
            _full_spec((1, 16)),
            _full_spec((1, 16)),
            _full_spec((16, _C)),
            _full_spec((1, _C)),
            _full_spec((1, 16)),
            _full_spec((1, 16)),
            _full_spec((16, 16)),
            _full_spec((1, 16)),
            _full_spec((16, _C)),
        ],
        out_specs=pl.BlockSpec((_BN, _C), lambda i: (i, 0)),
        out_shape=jax.ShapeDtypeStruct((_N, _C), jnp.float32),
    )(w1_e3, xv_e3, pp_e3, pc3, *params)


def kernel(p, x, o, Wq, bq, Wk, bk, Wv, bv, Wp1, bp1, g_p, be_p, Wp2, bp2,
           g_w1, be_w1, Ww1, bw1, g_w2, be_w2, Ww2, bw2):
    f32 = jnp.float32
    W = jnp.concatenate([Wq, Wk, Wv], axis=1)
    b = jnp.concatenate([bq, bk, bv])[None, :]
    q, xk, xv = _qkv(x, W, b)

    p8 = jnp.pad(p, ((0, 0), (0, 5)))
    p_pad = jnp.pad(p8, ((0, _NPAD - _N), (0, 0)), constant_values=1e9)
    pt3 = p_pad.T.reshape(8, _NCHUNK, _CB).transpose(1, 0, 2)
    idx = _knn(p_pad, pt3)
    idx_flat = idx[:_N].reshape(-1)

    pp128 = jnp.pad(p, ((0, 0), (0, _C - 3)))
    xk_e, xv_e, pp_e = _gather3(idx_flat, xk, xv, pp128)

    Wp1p = jnp.zeros((_C, 16), f32).at[:3, :3].set(Wp1)
    bp1p = jnp.zeros((1, 16), f32).at[0, :3].set(bp1)
    M = float(_E)
    pp_e3 = pp_e.reshape(_N, _NS, _C)
    pc3 = pp128.reshape(_N, 1, _C)
    ts = _tstats(pp_e3, pc3, Wp1p, bp1p)
    mu_t = ts[0] / M
    var_t = ts[1] / M - mu_t * mu_t
    gp = jnp.zeros((16,), f32).at[:3].set(g_p)
    bep = jnp.zeros((16,), f32).at[:3].set(be_p)
    a_t = (gp * lax.rsqrt(var_t + _EPS))[None, :]
    b_t = bep[None, :] - mu_t[None, :] * a_t

    Wp2p = jnp.zeros((16, _C), f32).at[:3, :].set(Wp2)
    bp2r = bp2[None, :]
    xk_e3 = xk_e.reshape(_N, _NS, _C)
    xv_e3 = xv_e.reshape(_N, _NS, _C)
    q3 = q.reshape(_N, 1, _C)
    front_params = (Wp1p, bp1p, a_t, b_t, Wp2p, bp2r)

    ws = _w0stats(pp_e3, pc3, xk_e3, q3, *front_params)
    mu_w = ws[0] / M
    var_w = ws[1] / M - mu_w * mu_w
    a_w = (g_w1 * lax.rsqrt(var_w + _EPS))[None, :]
    b_w = be_w1[None, :] - mu_w[None, :] * a_w

    w1_e3, w1s = _w1pass(pp_e3, pc3, xk_e3, q3, *front_params,
                         a_w, b_w, Ww1, bw1[None, :])
    mu1 = w1s[0] / M
    var1 = w1s[1] / M - mu1 * mu1
    a_2 = (g_w2 * lax.rsqrt(var1 + _EPS))[None, :]
    b_2 = be_w2[None, :] - mu1[None, :] * a_2

    tmat = (lax.broadcasted_iota(jnp.int32, (16, _C), 1) % 16 ==
            lax.broadcasted_iota(jnp.int32, (16, _C), 0)).astype(f32)
    out = _final(w1_e3, xv_e3, pp_e3, pc3, *front_params,
                 a_2, b_2, Ww2, bw2[None, :], tmat)
    return out
